# baseline (speedup 1.0000x reference)
import jax
import jax.numpy as jnp
from jax.experimental import pallas as pl
from jax.experimental.pallas import tpu as pltpu

_TM = 384
_VMEM_LIMIT_BYTES = 60 * 1024 * 1024


def _rotate_kernel(x_ref, w_ref, o_ref):
    o_ref[...] = jnp.dot(
        x_ref[...].astype(jnp.bfloat16),
        w_ref[...],
        preferred_element_type=jnp.float32,
    )


@jax.jit
def kernel(x, weight):
    H = weight.shape[0]
    lead = x.shape[:-1]
    x2d = x.astype(jnp.float32).reshape(-1, H)
    M = x2d.shape[0]

    tm = min(_TM, M)
    pad = (-M) % tm
    if pad:
        x2d = jnp.pad(x2d, ((0, pad), (0, 0)))
    Mp = x2d.shape[0]

    w_bf16 = weight.astype(jnp.bfloat16)

    out = pl.pallas_call(
        _rotate_kernel,
        out_shape=jax.ShapeDtypeStruct((Mp, H), jnp.float32),
        grid=(Mp // tm,),
        in_specs=[
            pl.BlockSpec((tm, H), lambda i: (i, 0)),
            pl.BlockSpec((H, H), lambda i: (0, 0)),
        ],
        out_specs=pl.BlockSpec((tm, H), lambda i: (i, 0)),
        compiler_params=pltpu.CompilerParams(
            dimension_semantics=("parallel",),
            vmem_limit_bytes=_VMEM_LIMIT_BYTES,
        ),
        cost_estimate=pl.CostEstimate(
            flops=2 * Mp * H * H,
            bytes_accessed=4 * Mp * H + 2 * H * H + 4 * Mp * H,
            transcendentals=0,
        ),
    )(x2d, w_bf16)

    if pad:
        out = out[:M]
    return out.reshape(lead + (H,))


# speedup vs baseline: 1.5544x; 1.5544x over previous
import jax
import jax.numpy as jnp
from jax.experimental import pallas as pl
from jax.experimental.pallas import tpu as pltpu

_TM = 512
_NSPLIT = 2
_VMEM_LIMIT_BYTES = 60 * 1024 * 1024


def _rotate_kernel(x_ref, w_ref, o_ref):
    o_ref[...] = jnp.dot(
        x_ref[...].astype(jnp.bfloat16),
        w_ref[...],
        preferred_element_type=jnp.float32,
    )


@jax.jit
def kernel(x, weight):
    H = weight.shape[0]
    lead = x.shape[:-1]
    x2d = x.astype(jnp.float32).reshape(-1, H)
    M = x2d.shape[0]

    tm = _TM if M % _TM == 0 else M
    pad = (-M) % tm
    if pad:
        x2d = jnp.pad(x2d, ((0, pad), (0, 0)))
    Mp = x2d.shape[0]

    nsplit = _NSPLIT if H % _NSPLIT == 0 else 1
    tn = H // nsplit
    w_bf16 = weight.astype(jnp.bfloat16)

    out = pl.pallas_call(
        _rotate_kernel,
        out_shape=jax.ShapeDtypeStruct((Mp, H), jnp.float32),
        grid=(nsplit, Mp // tm),
        in_specs=[
            pl.BlockSpec((tm, H), lambda j, i: (i, 0)),
            pl.BlockSpec((H, tn), lambda j, i: (0, j)),
        ],
        out_specs=pl.BlockSpec((tm, tn), lambda j, i: (i, j)),
        compiler_params=pltpu.CompilerParams(
            dimension_semantics=("parallel", "parallel"),
            vmem_limit_bytes=_VMEM_LIMIT_BYTES,
        ),
        cost_estimate=pl.CostEstimate(
            flops=2 * Mp * H * H,
            bytes_accessed=2 * 4 * Mp * H + 2 * H * H + 4 * Mp * H,
            transcendentals=0,
        ),
    )(x2d, w_bf16)

    if pad:
        out = out[:M]
    return out.reshape(lead + (H,))
